# Initial kernel scaffold; baseline (speedup 1.0000x reference)
import jax
import jax.numpy as jnp
from jax.experimental import pallas as pl


def kernel(x, x_e, edge_index, batch, params):
    raise NotImplementedError("write your pallas kernel here")


# speedup vs baseline: 6.4207x; 6.4207x over previous
import functools

import jax
import jax.numpy as jnp
from jax import lax
from jax.experimental import pallas as pl
from jax.experimental.pallas import tpu as pltpu
from jax.experimental.pallas import tpu_sc as plsc

_N = 10000
_H = 32
_G = 64
_C = 10

_NC = 2
_NS = 16
_NW = _NC * _NS
_CHUNK = 128
_NCHUNK = 80
_EPW = _CHUNK * _NCHUNK
_EPAD = _EPW * _NW
_RPS = 632
_NPAD = _RPS * _NS


def _segsum_sc(y, src_w, dst_w, zrows):
    mesh = plsc.VectorSubcoreMesh(core_axis_name="c", subcore_axis_name="s")

    @functools.partial(
        pl.kernel,
        mesh=mesh,
        compiler_params=pltpu.CompilerParams(use_tc_tiling_on_sc=False),
        out_type=jax.ShapeDtypeStruct((_NC, _NPAD, _H), jnp.float32),
        scratch_types=[
            pltpu.VMEM((_NCHUNK, _CHUNK), jnp.int32),
            pltpu.VMEM((_NCHUNK, _CHUNK), jnp.int32),
            pltpu.VMEM((_CHUNK, _H), jnp.float32),
            pltpu.VMEM_SHARED((_NPAD, _H), jnp.float32),
            pltpu.SemaphoreType.DMA,
        ],
    )
    def k(y_hbm, src_hbm, dst_hbm, z_hbm, out_hbm, sidx, didx, rows, acc, sem):
        c = lax.axis_index("c")
        s = lax.axis_index("s")
        wid = s * _NC + c
        r0 = s * _RPS
        pltpu.sync_copy(z_hbm.at[pl.ds(r0, _RPS)], acc.at[pl.ds(r0, _RPS)])
        pltpu.sync_copy(src_hbm.at[wid], sidx)
        pltpu.sync_copy(dst_hbm.at[wid], didx)
        plsc.subcore_barrier()

        @pl.loop(0, _NCHUNK)
        def _(ci):
            pltpu.async_copy(y_hbm.at[sidx.at[ci]], rows, sem).wait()
            pltpu.sync_copy(rows, acc.at[didx.at[ci]], add=True)

        plsc.subcore_barrier()
        pltpu.sync_copy(acc.at[pl.ds(r0, _RPS)], out_hbm.at[c, pl.ds(r0, _RPS)])

    return k(y, src_w, dst_w, zrows)


def _leaky(v):
    return jnp.where(v >= 0, v, 0.01 * v)


def _proj(x, w):
    def body(x_ref, w_ref, o_ref):
        o_ref[...] = jnp.dot(x_ref[...], w_ref[...],
                             preferred_element_type=jnp.float32, precision=lax.Precision.HIGHEST)

    return pl.pallas_call(
        body,
        out_shape=jax.ShapeDtypeStruct((x.shape[0], w.shape[1]), jnp.float32),
    )(x, w)


def _mlp_block(t, agg, b1, gamma, beta, w2, b2, w_next):
    def body(t_ref, a_ref, b1_ref, g_ref, be_ref, w2_ref, b2_ref, *rest):
        pre = t_ref[...] + a_ref[0, :_N, :] + a_ref[1, :_N, :] + b1_ref[...]
        m = jnp.mean(pre, axis=0, keepdims=True)
        d = pre - m
        v = jnp.mean(d * d, axis=0, keepdims=True)
        hn = d * (g_ref[...] * lax.rsqrt(v + 1e-5)) + be_ref[...]
        hn = _leaky(hn)
        h = jnp.dot(hn, w2_ref[...], preferred_element_type=jnp.float32, precision=lax.Precision.HIGHEST)
        h = _leaky(h + b2_ref[...])
        if w_next is None:
            (o_ref,) = rest
            o_ref[...] = h
        else:
            wn_ref, o_ref, t2_ref = rest
            o_ref[...] = h
            t2_ref[...] = jnp.dot(h, wn_ref[...],
                                  preferred_element_type=jnp.float32, precision=lax.Precision.HIGHEST)

    n = t.shape[0]
    outs = [jax.ShapeDtypeStruct((n, _H), jnp.float32)]
    args = [t, agg, b1, gamma, beta, w2, b2]
    if w_next is not None:
        outs.append(jax.ShapeDtypeStruct((n, _H), jnp.float32))
        args.append(w_next)
    res = pl.pallas_call(body, out_shape=tuple(outs))(*args)
    return res if w_next is not None else (res[0], None)


def _head(h1, h2, h3, batch_row, w1, b1, w2, b2):
    def body(h1_ref, h2_ref, h3_ref, bt_ref, w1_ref, b1_ref, w2_ref, b2_ref,
             o_ref):
        hcat = jnp.concatenate(
            [h1_ref[...], h2_ref[...], h3_ref[...]], axis=1)
        seg = lax.broadcasted_iota(jnp.int32, (_G, _N), 0)
        onehot = (seg == bt_ref[...]).astype(jnp.float32)
        p = jnp.dot(onehot, hcat, preferred_element_type=jnp.float32, precision=lax.Precision.HIGHEST)
        z = jnp.dot(p, w1_ref[...], preferred_element_type=jnp.float32, precision=lax.Precision.HIGHEST)
        z = jnp.maximum(z + b1_ref[...], 0.0)
        z = jnp.dot(z, w2_ref[...], preferred_element_type=jnp.float32, precision=lax.Precision.HIGHEST)
        o_ref[...] = _leaky(z + b2_ref[...])

    return pl.pallas_call(
        body,
        out_shape=jax.ShapeDtypeStruct((_G, _C), jnp.float32),
    )(h1, h2, h3, batch_row, w1, b1, w2, b2)


def kernel(x, x_e, edge_index, batch, params):
    del x_e
    src = edge_index[0].astype(jnp.int32)
    dst = edge_index[1].astype(jnp.int32)
    npad = _EPAD - src.shape[0]
    src_w = jnp.concatenate(
        [src, jnp.zeros((npad,), jnp.int32)]).reshape(_NW, _NCHUNK, _CHUNK)
    dst_w = jnp.concatenate(
        [dst, jnp.full((npad,), _N, jnp.int32)]).reshape(_NW, _NCHUNK, _CHUNK)
    zrows = jnp.zeros((_NPAD, _H), jnp.float32)
    batch_row = batch.astype(jnp.int32).reshape(1, _N)

    p1, p2, p3 = params["conv1"], params["conv2"], params["conv3"]

    def r(v):
        return v.reshape(1, -1)

    t1 = _proj(x, p1["W1"])
    a1 = _segsum_sc(t1, src_w, dst_w, zrows)
    h1, t2 = _mlp_block(t1, a1, r(p1["b1"]), r(p1["gamma"]), r(p1["beta"]),
                        p1["W2"], r(p1["b2"]), p2["W1"])
    a2 = _segsum_sc(t2, src_w, dst_w, zrows)
    h2, t3 = _mlp_block(t2, a2, r(p2["b1"]), r(p2["gamma"]), r(p2["beta"]),
                        p2["W2"], r(p2["b2"]), p3["W1"])
    a3 = _segsum_sc(t3, src_w, dst_w, zrows)
    h3, _ = _mlp_block(t3, a3, r(p3["b1"]), r(p3["gamma"]), r(p3["beta"]),
                       p3["W2"], r(p3["b2"]), None)
    return _head(h1, h2, h3, batch_row,
                 params["lin1"]["W"], r(params["lin1"]["b"]),
                 params["lin2"]["W"], r(params["lin2"]["b"]))


# speedup vs baseline: 14.0528x; 2.1887x over previous
import functools

import jax
import jax.numpy as jnp
from jax import lax
from jax.experimental import pallas as pl
from jax.experimental.pallas import tpu as pltpu
from jax.experimental.pallas import tpu_sc as plsc

_N = 10000
_H = 32
_G = 64
_C = 10

_NC = 2
_NS = 16
_NW = _NC * _NS
_CHUNK = 128
_NCHUNK = 80
_EPW = _CHUNK * _NCHUNK
_EPAD = _EPW * _NW
_RPS = 632
_NPAD = _RPS * _NS


def _segsum_sc(y, src_w, dst_w, zrows):
    mesh = plsc.VectorSubcoreMesh(core_axis_name="c", subcore_axis_name="s")

    @functools.partial(
        pl.kernel,
        mesh=mesh,
        compiler_params=pltpu.CompilerParams(use_tc_tiling_on_sc=False),
        out_type=jax.ShapeDtypeStruct((_NC, _NPAD, _H), jnp.float32),
        scratch_types=[
            pltpu.VMEM((_NCHUNK, _CHUNK), jnp.int32),
            pltpu.VMEM((_NCHUNK, _CHUNK), jnp.int32),
            pltpu.VMEM((8, _CHUNK, _H), jnp.float32),
            pltpu.VMEM_SHARED((_NPAD, _H), jnp.float32),
            pltpu.VMEM_SHARED((_NPAD, _H), jnp.float32),
            pltpu.SemaphoreType.DMA,
            pltpu.SemaphoreType.DMA,
            pltpu.SemaphoreType.DMA,
            pltpu.SemaphoreType.DMA,
        ],
    )
    def k(y_hbm, src_hbm, dst_hbm, z_hbm, out_hbm, sidx, didx, rows, ys, acc,
          sg0, sg1, ss0, ss1):
        c = lax.axis_index("c")
        s = lax.axis_index("s")
        wid = s * _NC + c
        r0 = s * _RPS
        pltpu.sync_copy(z_hbm.at[pl.ds(r0, _RPS)], acc.at[pl.ds(r0, _RPS)])
        @pl.when(s < _NS - 1)
        def _():
            pltpu.sync_copy(y_hbm.at[pl.ds(r0, _RPS)], ys.at[pl.ds(r0, _RPS)])

        @pl.when(s == _NS - 1)
        def _():
            rem = _N - (_NS - 1) * _RPS
            pltpu.sync_copy(y_hbm.at[pl.ds((_NS - 1) * _RPS, rem)],
                            ys.at[pl.ds((_NS - 1) * _RPS, rem)])

        pltpu.sync_copy(src_hbm.at[wid], sidx)
        pltpu.sync_copy(dst_hbm.at[wid], didx)
        plsc.subcore_barrier()

        @pl.loop(0, _NCHUNK // 8)
        def _(u):
            c0 = u * 8
            hg0 = [pltpu.async_copy(ys.at[sidx.at[c0 + j]], rows.at[j], sg0)
                   for j in range(4)]
            hg1 = [pltpu.async_copy(ys.at[sidx.at[c0 + 4 + j]],
                                    rows.at[4 + j], sg1) for j in range(4)]
            for h in hg0:
                h.wait()
            hs0 = [pltpu.async_copy(rows.at[j], acc.at[didx.at[c0 + j]],
                                    ss0, add=True) for j in range(4)]
            for h in hg1:
                h.wait()
            hs1 = [pltpu.async_copy(rows.at[4 + j],
                                    acc.at[didx.at[c0 + 4 + j]],
                                    ss1, add=True) for j in range(4)]
            for h in hs0 + hs1:
                h.wait()

        plsc.subcore_barrier()
        pltpu.sync_copy(acc.at[pl.ds(r0, _RPS)], out_hbm.at[c, pl.ds(r0, _RPS)])

    return k(y, src_w, dst_w, zrows)


def _leaky(v):
    return jnp.where(v >= 0, v, 0.01 * v)


def _proj(x, w):
    def body(x_ref, w_ref, o_ref):
        o_ref[...] = jnp.dot(x_ref[...], w_ref[...],
                             preferred_element_type=jnp.float32, precision=lax.Precision.HIGHEST)

    return pl.pallas_call(
        body,
        out_shape=jax.ShapeDtypeStruct((x.shape[0], w.shape[1]), jnp.float32),
    )(x, w)


def _mlp_block(t, agg, b1, gamma, beta, w2, b2, w_next):
    def body(t_ref, a_ref, b1_ref, g_ref, be_ref, w2_ref, b2_ref, *rest):
        pre = t_ref[...] + a_ref[0, :_N, :] + a_ref[1, :_N, :] + b1_ref[...]
        m = jnp.mean(pre, axis=0, keepdims=True)
        d = pre - m
        v = jnp.mean(d * d, axis=0, keepdims=True)
        hn = d * (g_ref[...] * lax.rsqrt(v + 1e-5)) + be_ref[...]
        hn = _leaky(hn)
        h = jnp.dot(hn, w2_ref[...], preferred_element_type=jnp.float32, precision=lax.Precision.HIGHEST)
        h = _leaky(h + b2_ref[...])
        if w_next is None:
            (o_ref,) = rest
            o_ref[...] = h
        else:
            wn_ref, o_ref, t2_ref = rest
            o_ref[...] = h
            t2_ref[...] = jnp.dot(h, wn_ref[...],
                                  preferred_element_type=jnp.float32, precision=lax.Precision.HIGHEST)

    n = t.shape[0]
    outs = [jax.ShapeDtypeStruct((n, _H), jnp.float32)]
    args = [t, agg, b1, gamma, beta, w2, b2]
    if w_next is not None:
        outs.append(jax.ShapeDtypeStruct((n, _H), jnp.float32))
        args.append(w_next)
    res = pl.pallas_call(body, out_shape=tuple(outs))(*args)
    return res if w_next is not None else (res[0], None)


def _head(h1, h2, h3, batch_row, w1, b1, w2, b2):
    def body(h1_ref, h2_ref, h3_ref, bt_ref, w1_ref, b1_ref, w2_ref, b2_ref,
             o_ref):
        hcat = jnp.concatenate(
            [h1_ref[...], h2_ref[...], h3_ref[...]], axis=1)
        seg = lax.broadcasted_iota(jnp.int32, (_G, _N), 0)
        onehot = (seg == bt_ref[...]).astype(jnp.float32)
        p = jnp.dot(onehot, hcat, preferred_element_type=jnp.float32, precision=lax.Precision.HIGHEST)
        z = jnp.dot(p, w1_ref[...], preferred_element_type=jnp.float32, precision=lax.Precision.HIGHEST)
        z = jnp.maximum(z + b1_ref[...], 0.0)
        z = jnp.dot(z, w2_ref[...], preferred_element_type=jnp.float32, precision=lax.Precision.HIGHEST)
        o_ref[...] = _leaky(z + b2_ref[...])

    return pl.pallas_call(
        body,
        out_shape=jax.ShapeDtypeStruct((_G, _C), jnp.float32),
    )(h1, h2, h3, batch_row, w1, b1, w2, b2)


def kernel(x, x_e, edge_index, batch, params):
    del x_e
    src = edge_index[0].astype(jnp.int32)
    dst = edge_index[1].astype(jnp.int32)
    npad = _EPAD - src.shape[0]
    src_w = jnp.concatenate(
        [src, jnp.zeros((npad,), jnp.int32)]).reshape(_NW, _NCHUNK, _CHUNK)
    dst_w = jnp.concatenate(
        [dst, jnp.full((npad,), _N, jnp.int32)]).reshape(_NW, _NCHUNK, _CHUNK)
    zrows = jnp.zeros((_NPAD, _H), jnp.float32)
    batch_row = batch.astype(jnp.int32).reshape(1, _N)

    p1, p2, p3 = params["conv1"], params["conv2"], params["conv3"]

    def r(v):
        return v.reshape(1, -1)

    t1 = _proj(x, p1["W1"])
    a1 = _segsum_sc(t1, src_w, dst_w, zrows)
    h1, t2 = _mlp_block(t1, a1, r(p1["b1"]), r(p1["gamma"]), r(p1["beta"]),
                        p1["W2"], r(p1["b2"]), p2["W1"])
    a2 = _segsum_sc(t2, src_w, dst_w, zrows)
    h2, t3 = _mlp_block(t2, a2, r(p2["b1"]), r(p2["gamma"]), r(p2["beta"]),
                        p2["W2"], r(p2["b2"]), p3["W1"])
    a3 = _segsum_sc(t3, src_w, dst_w, zrows)
    h3, _ = _mlp_block(t3, a3, r(p3["b1"]), r(p3["gamma"]), r(p3["beta"]),
                       p3["W2"], r(p3["b2"]), None)
    return _head(h1, h2, h3, batch_row,
                 params["lin1"]["W"], r(params["lin1"]["b"]),
                 params["lin2"]["W"], r(params["lin2"]["b"]))


# speedup vs baseline: 17.3357x; 1.2336x over previous
import functools

import jax
import jax.numpy as jnp
from jax import lax
from jax.experimental import pallas as pl
from jax.experimental.pallas import tpu as pltpu
from jax.experimental.pallas import tpu_sc as plsc

_N = 10000
_H = 32
_G = 64
_C = 10

_NC = 2
_NS = 16
_NW = _NC * _NS
_CHUNK = 128
_NCHUNK = 80
_EPW = _CHUNK * _NCHUNK
_EPAD = _EPW * _NW
_NPAD = 10112
_PR = _N // 4
_PRPAD = _NPAD // 4


def _segsum_sc(yp, src_w, dst_w, z):
    mesh = plsc.VectorSubcoreMesh(core_axis_name="c", subcore_axis_name="s")

    @functools.partial(
        pl.kernel,
        mesh=mesh,
        compiler_params=pltpu.CompilerParams(use_tc_tiling_on_sc=False),
        out_type=jax.ShapeDtypeStruct((_NC, _PRPAD, 128), jnp.float32),
        scratch_types=[
            pltpu.VMEM((_NCHUNK, _CHUNK), jnp.int32),
            pltpu.VMEM((_NCHUNK, _CHUNK), jnp.int32),
            pltpu.VMEM((8, _CHUNK, _H), jnp.float32),
            pltpu.VMEM_SHARED((_NPAD, _H), jnp.float32),
            pltpu.VMEM_SHARED((_NPAD, _H), jnp.float32),
            pltpu.SemaphoreType.DMA,
            pltpu.SemaphoreType.DMA,
            pltpu.SemaphoreType.DMA,
            pltpu.SemaphoreType.DMA,
        ],
    )
    def k(y_hbm, src_hbm, dst_hbm, z_hbm, out_hbm, sidx, didx, rows, ys, acc,
          sg0, sg1, ss0, ss1):
        c = lax.axis_index("c")
        s = lax.axis_index("s")
        wid = s * _NC + c
        pltpu.sync_copy(z_hbm.at[pl.ds(s * (_NPAD // _NS), _NPAD // _NS)],
                        acc.at[pl.ds(s * (_NPAD // _NS), _NPAD // _NS)])
        for kk in range(4):
            @pl.when(s < _NS - 1)
            def _():
                pltpu.sync_copy(
                    y_hbm.at[pl.ds(s * 160, 160), pl.ds(32 * kk, 32)],
                    ys.at[pl.ds(kk * _PRPAD + s * 160, 160)])

            @pl.when(s == _NS - 1)
            def _():
                pltpu.sync_copy(
                    y_hbm.at[pl.ds(2400, _PR - 2400), pl.ds(32 * kk, 32)],
                    ys.at[pl.ds(kk * _PRPAD + 2400, _PR - 2400)])

        pltpu.sync_copy(src_hbm.at[wid], sidx)
        pltpu.sync_copy(dst_hbm.at[wid], didx)
        plsc.subcore_barrier()

        @pl.loop(0, _NCHUNK // 8)
        def _(u):
            c0 = u * 8
            hg0 = [pltpu.async_copy(ys.at[sidx.at[c0 + j]], rows.at[j], sg0)
                   for j in range(4)]
            hg1 = [pltpu.async_copy(ys.at[sidx.at[c0 + 4 + j]],
                                    rows.at[4 + j], sg1) for j in range(4)]
            for h in hg0:
                h.wait()
            hs0 = [pltpu.async_copy(rows.at[j], acc.at[didx.at[c0 + j]],
                                    ss0, add=True) for j in range(4)]
            for h in hg1:
                h.wait()
            hs1 = [pltpu.async_copy(rows.at[4 + j],
                                    acc.at[didx.at[c0 + 4 + j]],
                                    ss1, add=True) for j in range(4)]
            for h in hs0 + hs1:
                h.wait()

        plsc.subcore_barrier()

        for kk in range(4):
            @pl.when(s < _NS - 1)
            def _():
                pltpu.sync_copy(
                    acc.at[pl.ds(kk * _PRPAD + s * 160, 160)],
                    out_hbm.at[c, pl.ds(s * 160, 160), pl.ds(32 * kk, 32)])

            @pl.when(s == _NS - 1)
            def _():
                pltpu.sync_copy(
                    acc.at[pl.ds(kk * _PRPAD + 2400, _PRPAD - 2400)],
                    out_hbm.at[c, pl.ds(2400, _PRPAD - 2400),
                               pl.ds(32 * kk, 32)])

    return k(yp, src_w, dst_w, z)


def _leaky(v):
    return jnp.where(v >= 0, v, 0.01 * v)


def _dot(a, b):
    return jnp.dot(a, b, preferred_element_type=jnp.float32,
                   precision=lax.Precision.HIGHEST)


def _fold4(v):
    return (v[:, 0:32] + v[:, 32:64] + v[:, 64:96] + v[:, 96:128]) * 0.25


def _tile4(v):
    return jnp.concatenate([v, v, v, v], axis=1)


def _proj(x4, w1x4):
    def body(x_ref, w_ref, o_ref):
        o_ref[...] = _dot(x_ref[...], w_ref[...])

    return pl.pallas_call(
        body,
        out_shape=jax.ShapeDtypeStruct((_PR, 128), jnp.float32),
    )(x4, w1x4)


def _mlp_block(t, agg, b1, gamma, beta, w2x4, b2, wnx4):
    def body(t_ref, a_ref, b1_ref, g_ref, be_ref, w2_ref, b2_ref, *rest):
        pre = t_ref[...] + a_ref[0, :_PR, :] + a_ref[1, :_PR, :] + b1_ref[...]
        m = _tile4(_fold4(jnp.mean(pre, axis=0, keepdims=True)))
        d = pre - m
        v = _tile4(_fold4(jnp.mean(d * d, axis=0, keepdims=True)))
        hn = d * (g_ref[...] * lax.rsqrt(v + 1e-5)) + be_ref[...]
        hn = _leaky(hn)
        h = _leaky(_dot(hn, w2_ref[...]) + b2_ref[...])
        if wnx4 is None:
            (o_ref,) = rest
            o_ref[...] = h
        else:
            wn_ref, o_ref, t2_ref = rest
            o_ref[...] = h
            t2_ref[...] = _dot(h, wn_ref[...])

    outs = [jax.ShapeDtypeStruct((_PR, 128), jnp.float32)]
    args = [t, agg, b1, gamma, beta, w2x4, b2]
    if wnx4 is not None:
        outs.append(jax.ShapeDtypeStruct((_PR, 128), jnp.float32))
        args.append(wnx4)
    res = pl.pallas_call(body, out_shape=tuple(outs))(*args)
    return res if wnx4 is not None else (res[0], None)


def _head(h1, h2, h3, batch4, w1, b1, w2, b2):
    def body(h1_ref, h2_ref, h3_ref, bt_ref, w1_ref, b1_ref, w2_ref, b2_ref,
             o_ref):
        seg = lax.broadcasted_iota(jnp.int32, (_G, _PR), 0)
        ps = []
        for h_ref in (h1_ref, h2_ref, h3_ref):
            hv = h_ref[...]
            p = jnp.zeros((_G, _H), jnp.float32)
            for kk in range(4):
                onehot = (seg == bt_ref[kk, :]).astype(jnp.float32)
                p = p + _dot(onehot, hv[:, 32 * kk:32 * (kk + 1)])
            ps.append(p)
        p = jnp.concatenate(ps, axis=1)
        z = jnp.maximum(_dot(p, w1_ref[...]) + b1_ref[...], 0.0)
        z = _dot(z, w2_ref[...]) + b2_ref[...]
        o_ref[...] = _leaky(z)

    return pl.pallas_call(
        body,
        out_shape=jax.ShapeDtypeStruct((_G, _C), jnp.float32),
    )(h1, h2, h3, batch4, w1, b1, w2, b2)


def _blockdiag4(w):
    a, b = w.shape
    z = jnp.zeros((a, b), w.dtype)
    return jnp.block([[w if i == j else z for j in range(4)]
                      for i in range(4)])


def kernel(x, x_e, edge_index, batch, params):
    del x_e
    src = edge_index[0].astype(jnp.int32)
    dst = edge_index[1].astype(jnp.int32)
    srcp = (src & 3) * _PRPAD + (src >> 2)
    dstp = (dst & 3) * _PRPAD + (dst >> 2)
    npad = _EPAD - src.shape[0]
    src_w = jnp.concatenate(
        [srcp, jnp.zeros((npad,), jnp.int32)]).reshape(_NW, _NCHUNK, _CHUNK)
    dst_w = jnp.concatenate(
        [dstp, jnp.full((npad,), _PR, jnp.int32)]).reshape(_NW, _NCHUNK, _CHUNK)
    z = jnp.zeros((_NPAD, _H), jnp.float32)
    batch4 = batch.astype(jnp.int32).reshape(_PR, 4).T
    x4 = x.reshape(_PR, 4 * 128)

    p1, p2, p3 = params["conv1"], params["conv2"], params["conv3"]

    def r4(v):
        return jnp.tile(v.reshape(1, -1), (1, 4))

    t1 = _proj(x4, _blockdiag4(p1["W1"]))
    a1 = _segsum_sc(t1, src_w, dst_w, z)
    h1, t2 = _mlp_block(t1, a1, r4(p1["b1"]), r4(p1["gamma"]),
                        r4(p1["beta"]), _blockdiag4(p1["W2"]), r4(p1["b2"]),
                        _blockdiag4(p2["W1"]))
    a2 = _segsum_sc(t2, src_w, dst_w, z)
    h2, t3 = _mlp_block(t2, a2, r4(p2["b1"]), r4(p2["gamma"]),
                        r4(p2["beta"]), _blockdiag4(p2["W2"]), r4(p2["b2"]),
                        _blockdiag4(p3["W1"]))
    a3 = _segsum_sc(t3, src_w, dst_w, z)
    h3, _ = _mlp_block(t3, a3, r4(p3["b1"]), r4(p3["gamma"]),
                       r4(p3["beta"]), _blockdiag4(p3["W2"]), r4(p3["b2"]),
                       None)
    return _head(h1, h2, h3, batch4,
                 params["lin1"]["W"], params["lin1"]["b"].reshape(1, -1),
                 params["lin2"]["W"], params["lin2"]["b"].reshape(1, -1))


# speedup vs baseline: 19.4277x; 1.1207x over previous
import functools

import jax
import jax.numpy as jnp
from jax import lax
from jax.experimental import pallas as pl
from jax.experimental.pallas import tpu as pltpu
from jax.experimental.pallas import tpu_sc as plsc

_N = 10000
_H = 32
_G = 64
_C = 10

_NC = 2
_NS = 16
_NW = _NC * _NS
_CHUNK = 128
_NCHUNK = 80
_EPW = _CHUNK * _NCHUNK
_EPAD = _EPW * _NW
_NPAD = 10112
_PR = _N // 4
_PRPAD = _NPAD // 4


def _segsum_sc(yp, src_w, dst_w, z):
    mesh = plsc.VectorSubcoreMesh(core_axis_name="c", subcore_axis_name="s")

    @functools.partial(
        pl.kernel,
        mesh=mesh,
        compiler_params=pltpu.CompilerParams(use_tc_tiling_on_sc=False),
        out_type=jax.ShapeDtypeStruct((_NC, _PRPAD, 128), jnp.float32),
        scratch_types=[
            pltpu.VMEM((_NCHUNK, _CHUNK), jnp.int32),
            pltpu.VMEM((_NCHUNK, _CHUNK), jnp.int32),
            pltpu.VMEM((8, _CHUNK, _H), jnp.float32),
            pltpu.VMEM_SHARED((_NPAD, _H), jnp.float32),
            pltpu.VMEM_SHARED((_NPAD, _H), jnp.float32),
            pltpu.SemaphoreType.DMA,
            pltpu.SemaphoreType.DMA,
            pltpu.SemaphoreType.DMA,
            pltpu.SemaphoreType.DMA,
        ],
    )
    def k(y_hbm, src_hbm, dst_hbm, z_hbm, out_hbm, sidx, didx, rows, ys, acc,
          sg0, sg1, ss0, ss1):
        c = lax.axis_index("c")
        s = lax.axis_index("s")
        wid = s * _NC + c
        pltpu.sync_copy(z_hbm.at[pl.ds(s * (_NPAD // _NS), _NPAD // _NS)],
                        acc.at[pl.ds(s * (_NPAD // _NS), _NPAD // _NS)])
        for kk in range(4):
            @pl.when(s < _NS - 1)
            def _():
                pltpu.sync_copy(
                    y_hbm.at[pl.ds(s * 160, 160), pl.ds(32 * kk, 32)],
                    ys.at[pl.ds(kk * _PRPAD + s * 160, 160)])

            @pl.when(s == _NS - 1)
            def _():
                pltpu.sync_copy(
                    y_hbm.at[pl.ds(2400, _PR - 2400), pl.ds(32 * kk, 32)],
                    ys.at[pl.ds(kk * _PRPAD + 2400, _PR - 2400)])

        pltpu.sync_copy(src_hbm.at[pl.ds(wid * _NCHUNK, _NCHUNK)], sidx)
        pltpu.sync_copy(dst_hbm.at[pl.ds(wid * _NCHUNK, _NCHUNK)], didx)
        plsc.subcore_barrier()

        @pl.loop(0, _NCHUNK // 8)
        def _(u):
            c0 = u * 8
            hg0 = [pltpu.async_copy(ys.at[sidx.at[c0 + j]], rows.at[j], sg0)
                   for j in range(4)]
            hg1 = [pltpu.async_copy(ys.at[sidx.at[c0 + 4 + j]],
                                    rows.at[4 + j], sg1) for j in range(4)]
            for h in hg0:
                h.wait()
            hs0 = [pltpu.async_copy(rows.at[j], acc.at[didx.at[c0 + j]],
                                    ss0, add=True) for j in range(4)]
            for h in hg1:
                h.wait()
            hs1 = [pltpu.async_copy(rows.at[4 + j],
                                    acc.at[didx.at[c0 + 4 + j]],
                                    ss1, add=True) for j in range(4)]
            for h in hs0 + hs1:
                h.wait()

        plsc.subcore_barrier()

        for kk in range(4):
            @pl.when(s < _NS - 1)
            def _():
                pltpu.sync_copy(
                    acc.at[pl.ds(kk * _PRPAD + s * 160, 160)],
                    out_hbm.at[c, pl.ds(s * 160, 160), pl.ds(32 * kk, 32)])

            @pl.when(s == _NS - 1)
            def _():
                pltpu.sync_copy(
                    acc.at[pl.ds(kk * _PRPAD + 2400, _PRPAD - 2400)],
                    out_hbm.at[c, pl.ds(2400, _PRPAD - 2400),
                               pl.ds(32 * kk, 32)])

    return k(yp, src_w, dst_w, z)


def _leaky(v):
    return jnp.where(v >= 0, v, 0.01 * v)


def _dot(a, b):
    return jnp.dot(a, b, preferred_element_type=jnp.float32)


def _fold4(v):
    return (v[:, 0:32] + v[:, 32:64] + v[:, 64:96] + v[:, 96:128]) * 0.25


def _tile4(v):
    return jnp.concatenate([v, v, v, v], axis=1)


def _proj(x4, w1x4):
    def body(x_ref, w_ref, o_ref):
        o_ref[...] = _dot(x_ref[...], w_ref[...])

    return pl.pallas_call(
        body,
        out_shape=jax.ShapeDtypeStruct((_PR, 128), jnp.float32),
    )(x4, w1x4)


def _mlp_block(t, agg, b1, gamma, beta, w2x4, b2, wnx4):
    def body(t_ref, a_ref, b1_ref, g_ref, be_ref, w2_ref, b2_ref, wn_ref,
             o_ref, t2_ref):
        pre = t_ref[...] + a_ref[0, :_PR, :] + a_ref[1, :_PR, :] + b1_ref[...]
        m = _tile4(_fold4(jnp.mean(pre, axis=0, keepdims=True)))
        d = pre - m
        v = _tile4(_fold4(jnp.mean(d * d, axis=0, keepdims=True)))
        hn = d * (g_ref[...] * lax.rsqrt(v + 1e-5)) + be_ref[...]
        hn = _leaky(hn)
        h = _leaky(_dot(hn, w2_ref[...]) + b2_ref[...])
        o_ref[...] = h
        t2_ref[...] = _dot(h, wn_ref[...])

    outs = (jax.ShapeDtypeStruct((_PR, 128), jnp.float32),
            jax.ShapeDtypeStruct((_PR, 128), jnp.float32))
    return pl.pallas_call(body, out_shape=outs)(
        t, agg, b1, gamma, beta, w2x4, b2, wnx4)


def _mlp3_head(t, agg, b1, gamma, beta, w2x4, b2, h1, h2, batch4,
               lw1, lb1, lw2, lb2):
    def body(t_ref, a_ref, b1_ref, g_ref, be_ref, w2_ref, b2_ref,
             h1_ref, h2_ref, bt_ref, lw1_ref, lb1_ref, lw2_ref, lb2_ref,
             o_ref):
        pre = t_ref[...] + a_ref[0, :_PR, :] + a_ref[1, :_PR, :] + b1_ref[...]
        m = _tile4(_fold4(jnp.mean(pre, axis=0, keepdims=True)))
        d = pre - m
        v = _tile4(_fold4(jnp.mean(d * d, axis=0, keepdims=True)))
        hn = d * (g_ref[...] * lax.rsqrt(v + 1e-5)) + be_ref[...]
        hn = _leaky(hn)
        h3 = _leaky(_dot(hn, w2_ref[...]) + b2_ref[...])
        seg = lax.broadcasted_iota(jnp.int32, (_G, _PR), 0)
        ps = []
        for hv in (h1_ref[...], h2_ref[...], h3):
            p = jnp.zeros((_G, _H), jnp.float32)
            for kk in range(4):
                onehot = (seg == bt_ref[kk, :]).astype(jnp.float32)
                p = p + _dot(onehot, hv[:, 32 * kk:32 * (kk + 1)])
            ps.append(p)
        p = jnp.concatenate(ps, axis=1)
        z = jnp.maximum(_dot(p, lw1_ref[...]) + lb1_ref[...], 0.0)
        z = _dot(z, lw2_ref[...]) + lb2_ref[...]
        o_ref[...] = _leaky(z)

    return pl.pallas_call(
        body,
        out_shape=jax.ShapeDtypeStruct((_G, _C), jnp.float32),
    )(t, agg, b1, gamma, beta, w2x4, b2, h1, h2, batch4, lw1, lb1, lw2, lb2)


def _blockdiag4(w):
    a, b = w.shape
    z = jnp.zeros((a, b), w.dtype)
    return jnp.block([[w if i == j else z for j in range(4)]
                      for i in range(4)])


def kernel(x, x_e, edge_index, batch, params):
    del x_e
    src = edge_index[0].astype(jnp.int32)
    dst = edge_index[1].astype(jnp.int32)
    srcp = (src & 3) * _PRPAD + (src >> 2)
    dstp = (dst & 3) * _PRPAD + (dst >> 2)
    npad = _EPAD - src.shape[0]
    src_w = jnp.pad(srcp, (0, npad)).reshape(_NW * _NCHUNK, _CHUNK)
    dst_w = jnp.pad(dstp, (0, npad),
                    constant_values=_PR).reshape(_NW * _NCHUNK, _CHUNK)
    z = jnp.zeros((_NPAD, _H), jnp.float32)
    batch4 = batch.astype(jnp.int32).reshape(_PR, 4).T
    x4 = x.reshape(_PR, 4 * 128)

    p1, p2, p3 = params["conv1"], params["conv2"], params["conv3"]

    def r4(v):
        return jnp.tile(v.reshape(1, -1), (1, 4))

    t1 = _proj(x4, _blockdiag4(p1["W1"]))
    a1 = _segsum_sc(t1, src_w, dst_w, z)
    h1, t2 = _mlp_block(t1, a1, r4(p1["b1"]), r4(p1["gamma"]),
                        r4(p1["beta"]), _blockdiag4(p1["W2"]), r4(p1["b2"]),
                        _blockdiag4(p2["W1"]))
    a2 = _segsum_sc(t2, src_w, dst_w, z)
    h2, t3 = _mlp_block(t2, a2, r4(p2["b1"]), r4(p2["gamma"]),
                        r4(p2["beta"]), _blockdiag4(p2["W2"]), r4(p2["b2"]),
                        _blockdiag4(p3["W1"]))
    a3 = _segsum_sc(t3, src_w, dst_w, z)
    return _mlp3_head(t3, a3, r4(p3["b1"]), r4(p3["gamma"]), r4(p3["beta"]),
                      _blockdiag4(p3["W2"]), r4(p3["b2"]), h1, h2, batch4,
                      params["lin1"]["W"], params["lin1"]["b"].reshape(1, -1),
                      params["lin2"]["W"], params["lin2"]["b"].reshape(1, -1))


# speedup vs baseline: 20.3425x; 1.0471x over previous
import functools

import jax
import jax.numpy as jnp
from jax import lax
from jax.experimental import pallas as pl
from jax.experimental.pallas import tpu as pltpu
from jax.experimental.pallas import tpu_sc as plsc

_N = 10000
_H = 32
_G = 64
_C = 10

_NC = 2
_NS = 16
_NW = _NC * _NS
_CHUNK = 128
_NCHUNK = 80
_EPW = _CHUNK * _NCHUNK
_EPAD = _EPW * _NW
_NPAD = 10112
_PR = _N // 4
_PRPAD = _NPAD // 4


def _segsum_sc(yp, src_w, dst_w, z):
    mesh = plsc.VectorSubcoreMesh(core_axis_name="c", subcore_axis_name="s")

    @functools.partial(
        pl.kernel,
        mesh=mesh,
        compiler_params=pltpu.CompilerParams(use_tc_tiling_on_sc=False),
        out_type=jax.ShapeDtypeStruct((_NC, _PRPAD, 128), jnp.float32),
        scratch_types=[
            pltpu.VMEM((_NCHUNK, _CHUNK), jnp.int32),
            pltpu.VMEM((_NCHUNK, _CHUNK), jnp.int32),
            pltpu.VMEM((8, _CHUNK, _H), jnp.float32),
            pltpu.VMEM_SHARED((_NPAD, _H), jnp.float32),
            pltpu.VMEM_SHARED((_NPAD, _H), jnp.float32),
            pltpu.SemaphoreType.DMA,
            pltpu.SemaphoreType.DMA,
            pltpu.SemaphoreType.DMA,
            pltpu.SemaphoreType.DMA,
        ],
    )
    def k(y_hbm, src_hbm, dst_hbm, z_hbm, out_hbm, sidx, didx, rows, ys, acc,
          sg0, sg1, ss0, ss1):
        c = lax.axis_index("c")
        s = lax.axis_index("s")
        wid = s * _NC + c
        hs = [pltpu.async_copy(
                  z_hbm.at[pl.ds(s * (_NPAD // _NS), _NPAD // _NS)],
                  acc.at[pl.ds(s * (_NPAD // _NS), _NPAD // _NS)], sg0),
              pltpu.async_copy(
                  src_hbm.at[pl.ds(wid * _NCHUNK, _NCHUNK)], sidx, ss0),
              pltpu.async_copy(
                  dst_hbm.at[pl.ds(wid * _NCHUNK, _NCHUNK)], didx, ss1)]

        @pl.when(s < _NS - 1)
        def _():
            hy = [pltpu.async_copy(
                      y_hbm.at[pl.ds(s * 160, 160), pl.ds(32 * kk, 32)],
                      ys.at[pl.ds(kk * _PRPAD + s * 160, 160)], sg1)
                  for kk in range(4)]
            for h in hy:
                h.wait()

        @pl.when(s == _NS - 1)
        def _():
            hy = [pltpu.async_copy(
                      y_hbm.at[pl.ds(2400, _PR - 2400), pl.ds(32 * kk, 32)],
                      ys.at[pl.ds(kk * _PRPAD + 2400, _PR - 2400)], sg1)
                  for kk in range(4)]
            for h in hy:
                h.wait()

        for h in hs:
            h.wait()
        plsc.subcore_barrier()

        @pl.loop(0, _NCHUNK // 8)
        def _(u):
            c0 = u * 8
            hg0 = [pltpu.async_copy(ys.at[sidx.at[c0 + j]], rows.at[j], sg0)
                   for j in range(4)]
            hg1 = [pltpu.async_copy(ys.at[sidx.at[c0 + 4 + j]],
                                    rows.at[4 + j], sg1) for j in range(4)]
            for h in hg0:
                h.wait()
            hs0 = [pltpu.async_copy(rows.at[j], acc.at[didx.at[c0 + j]],
                                    ss0, add=True) for j in range(4)]
            for h in hg1:
                h.wait()
            hs1 = [pltpu.async_copy(rows.at[4 + j],
                                    acc.at[didx.at[c0 + 4 + j]],
                                    ss1, add=True) for j in range(4)]
            for h in hs0 + hs1:
                h.wait()

        plsc.subcore_barrier()

        @pl.when(s < _NS - 1)
        def _():
            hw = [pltpu.async_copy(
                      acc.at[pl.ds(kk * _PRPAD + s * 160, 160)],
                      out_hbm.at[c, pl.ds(s * 160, 160), pl.ds(32 * kk, 32)],
                      sg0) for kk in range(4)]
            for h in hw:
                h.wait()

        @pl.when(s == _NS - 1)
        def _():
            hw = [pltpu.async_copy(
                      acc.at[pl.ds(kk * _PRPAD + 2400, _PRPAD - 2400)],
                      out_hbm.at[c, pl.ds(2400, _PRPAD - 2400),
                                 pl.ds(32 * kk, 32)], sg0)
                  for kk in range(4)]
            for h in hw:
                h.wait()

    return k(yp, src_w, dst_w, z)


def _leaky(v):
    return jnp.where(v >= 0, v, 0.01 * v)


def _dot(a, b):
    return jnp.dot(a, b, preferred_element_type=jnp.float32)


def _fold4(v):
    return (v[:, 0:32] + v[:, 32:64] + v[:, 64:96] + v[:, 96:128]) * 0.25


def _tile4(v):
    return jnp.concatenate([v, v, v, v], axis=1)


def _proj(x, w1):
    def body(x_ref, w_ref, o_ref):
        w = w_ref[...]
        for kk in range(4):
            o_ref[:, 32 * kk:32 * (kk + 1)] = _dot(
                x_ref[_PR * kk:_PR * (kk + 1), :], w)

    return pl.pallas_call(
        body,
        out_shape=jax.ShapeDtypeStruct((_PR, 128), jnp.float32),
    )(x, w1)


def _mlp_block(t, agg, b1, gamma, beta, w2x4, b2, wnx4):
    def body(t_ref, a_ref, b1_ref, g_ref, be_ref, w2_ref, b2_ref, wn_ref,
             o_ref, t2_ref):
        pre = t_ref[...] + a_ref[0, :_PR, :] + a_ref[1, :_PR, :] + b1_ref[...]
        m = _tile4(_fold4(jnp.mean(pre, axis=0, keepdims=True)))
        d = pre - m
        v = _tile4(_fold4(jnp.mean(d * d, axis=0, keepdims=True)))
        hn = d * (g_ref[...] * lax.rsqrt(v + 1e-5)) + be_ref[...]
        hn = _leaky(hn)
        h = _leaky(_dot(hn, w2_ref[...]) + b2_ref[...])
        o_ref[...] = h
        t2_ref[...] = _dot(h, wn_ref[...])

    outs = (jax.ShapeDtypeStruct((_PR, 128), jnp.float32),
            jax.ShapeDtypeStruct((_PR, 128), jnp.float32))
    return pl.pallas_call(body, out_shape=outs)(
        t, agg, b1, gamma, beta, w2x4, b2, wnx4)


def _mlp3_head(t, agg, b1, gamma, beta, w2x4, b2, h1, h2, batch4,
               lw1, lb1, lw2, lb2):
    def body(t_ref, a_ref, b1_ref, g_ref, be_ref, w2_ref, b2_ref,
             h1_ref, h2_ref, bt_ref, lw1_ref, lb1_ref, lw2_ref, lb2_ref,
             o_ref):
        pre = t_ref[...] + a_ref[0, :_PR, :] + a_ref[1, :_PR, :] + b1_ref[...]
        m = _tile4(_fold4(jnp.mean(pre, axis=0, keepdims=True)))
        d = pre - m
        v = _tile4(_fold4(jnp.mean(d * d, axis=0, keepdims=True)))
        hn = d * (g_ref[...] * lax.rsqrt(v + 1e-5)) + be_ref[...]
        hn = _leaky(hn)
        h3 = _leaky(_dot(hn, w2_ref[...]) + b2_ref[...])
        seg = lax.broadcasted_iota(jnp.int32, (_G, _PR), 0)
        ps = []
        for hv in (h1_ref[...], h2_ref[...], h3):
            p = jnp.zeros((_G, _H), jnp.float32)
            for kk in range(4):
                onehot = (seg == bt_ref[kk, :]).astype(jnp.float32)
                p = p + _dot(onehot, hv[:, 32 * kk:32 * (kk + 1)])
            ps.append(p)
        p = jnp.concatenate(ps, axis=1)
        z = jnp.maximum(_dot(p, lw1_ref[...]) + lb1_ref[...], 0.0)
        z = _dot(z, lw2_ref[...]) + lb2_ref[...]
        o_ref[...] = _leaky(z)

    return pl.pallas_call(
        body,
        out_shape=jax.ShapeDtypeStruct((_G, _C), jnp.float32),
    )(t, agg, b1, gamma, beta, w2x4, b2, h1, h2, batch4, lw1, lb1, lw2, lb2)


def _blockdiag4(w):
    a, b = w.shape
    z = jnp.zeros((a, b), w.dtype)
    return jnp.block([[w if i == j else z for j in range(4)]
                      for i in range(4)])


def kernel(x, x_e, edge_index, batch, params):
    del x_e
    src = edge_index[0].astype(jnp.int32)
    dst = edge_index[1].astype(jnp.int32)
    srcp = src + 28 * (src // _PR)
    dstp = dst + 28 * (dst // _PR)
    npad = _EPAD - src.shape[0]
    src_w = jnp.pad(srcp, (0, npad)).reshape(_NW * _NCHUNK, _CHUNK)
    dst_w = jnp.pad(dstp, (0, npad),
                    constant_values=_PR).reshape(_NW * _NCHUNK, _CHUNK)
    z = jnp.zeros((_NPAD, _H), jnp.float32)
    batch4 = batch.astype(jnp.int32).reshape(4, _PR)

    p1, p2, p3 = params["conv1"], params["conv2"], params["conv3"]

    def r4(v):
        return jnp.tile(v.reshape(1, -1), (1, 4))

    t1 = _proj(x, p1["W1"])
    a1 = _segsum_sc(t1, src_w, dst_w, z)
    h1, t2 = _mlp_block(t1, a1, r4(p1["b1"]), r4(p1["gamma"]),
                        r4(p1["beta"]), _blockdiag4(p1["W2"]), r4(p1["b2"]),
                        _blockdiag4(p2["W1"]))
    a2 = _segsum_sc(t2, src_w, dst_w, z)
    h2, t3 = _mlp_block(t2, a2, r4(p2["b1"]), r4(p2["gamma"]),
                        r4(p2["beta"]), _blockdiag4(p2["W2"]), r4(p2["b2"]),
                        _blockdiag4(p3["W1"]))
    a3 = _segsum_sc(t3, src_w, dst_w, z)
    return _mlp3_head(t3, a3, r4(p3["b1"]), r4(p3["gamma"]), r4(p3["beta"]),
                      _blockdiag4(p3["W2"]), r4(p3["b2"]), h1, h2, batch4,
                      params["lin1"]["W"], params["lin1"]["b"].reshape(1, -1),
                      params["lin2"]["W"], params["lin2"]["b"].reshape(1, -1))


# speedup vs baseline: 20.7551x; 1.0203x over previous
import functools

import jax
import jax.numpy as jnp
from jax import lax
from jax.experimental import pallas as pl
from jax.experimental.pallas import tpu as pltpu
from jax.experimental.pallas import tpu_sc as plsc

_N = 10000
_H = 32
_G = 64
_C = 10

_NC = 2
_NS = 16
_NW = _NC * _NS
_CHUNK = 128
_NCHUNK = 80
_EPW = _CHUNK * _NCHUNK
_EPAD = _EPW * _NW
_PR = _N // 4
_NA = _N + 8
_OUTR = 2504


def _segsum_sc(yp, src_w, dst_w, z):
    mesh = plsc.VectorSubcoreMesh(core_axis_name="c", subcore_axis_name="s")

    @functools.partial(
        pl.kernel,
        mesh=mesh,
        compiler_params=pltpu.CompilerParams(use_tc_tiling_on_sc=False),
        out_type=jax.ShapeDtypeStruct((_NC, _OUTR, 128), jnp.float32),
        scratch_types=[
            pltpu.VMEM((_NCHUNK, _CHUNK), jnp.int32),
            pltpu.VMEM((_NCHUNK, _CHUNK), jnp.int32),
            pltpu.VMEM((8, _CHUNK, _H), jnp.float32),
            pltpu.VMEM_SHARED((_N, _H), jnp.float32),
            pltpu.VMEM_SHARED((_NA, _H), jnp.float32),
            pltpu.SemaphoreType.DMA,
            pltpu.SemaphoreType.DMA,
            pltpu.SemaphoreType.DMA,
            pltpu.SemaphoreType.DMA,
        ],
    )
    def k(y_hbm, src_hbm, dst_hbm, z_hbm, out_hbm, sidx, didx, rows, ys, acc,
          sg0, sg1, ss0, ss1):
        c = lax.axis_index("c")
        s = lax.axis_index("s")
        wid = s * _NC + c
        hs = [pltpu.async_copy(
                  src_hbm.at[pl.ds(wid * _NCHUNK, _NCHUNK)], sidx, ss0),
              pltpu.async_copy(
                  dst_hbm.at[pl.ds(wid * _NCHUNK, _NCHUNK)], didx, ss1)]

        @pl.when(s < _NS - 1)
        def _():
            hz = pltpu.async_copy(z_hbm.at[pl.ds(s * 632, 632)],
                                  acc.at[pl.ds(s * 632, 632)], sg0)
            hy = [pltpu.async_copy(
                      y_hbm.at[pl.ds(s * 160, 160), pl.ds(32 * kk, 32)],
                      ys.at[pl.ds(kk * _PR + s * 160, 160)], sg1)
                  for kk in range(4)]
            hz.wait()
            for h in hy:
                h.wait()

        @pl.when(s == _NS - 1)
        def _():
            hz = pltpu.async_copy(z_hbm.at[pl.ds(9480, _NA - 9480)],
                                  acc.at[pl.ds(9480, _NA - 9480)], sg0)
            hy = [pltpu.async_copy(
                      y_hbm.at[pl.ds(2400, _PR - 2400), pl.ds(32 * kk, 32)],
                      ys.at[pl.ds(kk * _PR + 2400, _PR - 2400)], sg1)
                  for kk in range(4)]
            hz.wait()
            for h in hy:
                h.wait()

        for h in hs:
            h.wait()
        plsc.subcore_barrier()

        @pl.loop(0, _NCHUNK // 8)
        def _(u):
            c0 = u * 8
            hg0 = [pltpu.async_copy(ys.at[sidx.at[c0 + j]], rows.at[j], sg0)
                   for j in range(4)]
            hg1 = [pltpu.async_copy(ys.at[sidx.at[c0 + 4 + j]],
                                    rows.at[4 + j], sg1) for j in range(4)]
            for h in hg0:
                h.wait()
            hs0 = [pltpu.async_copy(rows.at[j], acc.at[didx.at[c0 + j]],
                                    ss0, add=True) for j in range(4)]
            for h in hg1:
                h.wait()
            hs1 = [pltpu.async_copy(rows.at[4 + j],
                                    acc.at[didx.at[c0 + 4 + j]],
                                    ss1, add=True) for j in range(4)]
            for h in hs0 + hs1:
                h.wait()

        plsc.subcore_barrier()

        @pl.when(s < _NS - 1)
        def _():
            hw = [pltpu.async_copy(
                      acc.at[pl.ds(kk * _PR + s * 160, 160)],
                      out_hbm.at[c, pl.ds(s * 160, 160), pl.ds(32 * kk, 32)],
                      sg0) for kk in range(4)]
            for h in hw:
                h.wait()

        @pl.when(s == _NS - 1)
        def _():
            hw = [pltpu.async_copy(
                      acc.at[pl.ds(kk * _PR + 2400, _PR - 2400)],
                      out_hbm.at[c, pl.ds(2400, _PR - 2400),
                                 pl.ds(32 * kk, 32)], sg0)
                  for kk in range(4)]
            for h in hw:
                h.wait()

    return k(yp, src_w, dst_w, z)


def _leaky(v):
    return jnp.where(v >= 0, v, 0.01 * v)


def _dot(a, b):
    return jnp.dot(a, b, preferred_element_type=jnp.float32)


def _fold4(v):
    return (v[:, 0:32] + v[:, 32:64] + v[:, 64:96] + v[:, 96:128]) * 0.25


def _tile4(v):
    return jnp.concatenate([v, v, v, v], axis=1)


def _proj(x, w1):
    def body(x_ref, w_ref, o_ref):
        w = w_ref[...]
        for kk in range(4):
            o_ref[:, 32 * kk:32 * (kk + 1)] = _dot(
                x_ref[_PR * kk:_PR * (kk + 1), :], w)

    return pl.pallas_call(
        body,
        out_shape=jax.ShapeDtypeStruct((_PR, 128), jnp.float32),
    )(x, w1)


def _mlp_block(t, agg, b1, gamma, beta, w2x4, b2, wnx4):
    def body(t_ref, a_ref, b1_ref, g_ref, be_ref, w2_ref, b2_ref, wn_ref,
             o_ref, t2_ref):
        pre = t_ref[...] + a_ref[0, :_PR, :] + a_ref[1, :_PR, :] + b1_ref[...]
        m = _tile4(_fold4(jnp.mean(pre, axis=0, keepdims=True)))
        d = pre - m
        v = _tile4(_fold4(jnp.mean(d * d, axis=0, keepdims=True)))
        hn = d * (g_ref[...] * lax.rsqrt(v + 1e-5)) + be_ref[...]
        hn = _leaky(hn)
        h = _leaky(_dot(hn, w2_ref[...]) + b2_ref[...])
        o_ref[...] = h
        t2_ref[...] = _dot(h, wn_ref[...])

    outs = (jax.ShapeDtypeStruct((_PR, 128), jnp.float32),
            jax.ShapeDtypeStruct((_PR, 128), jnp.float32))
    return pl.pallas_call(body, out_shape=outs)(
        t, agg, b1, gamma, beta, w2x4, b2, wnx4)


def _mlp3_head(t, agg, b1, gamma, beta, w2x4, b2, h1, h2, batch4,
               lw1, lb1, lw2, lb2):
    def body(t_ref, a_ref, b1_ref, g_ref, be_ref, w2_ref, b2_ref,
             h1_ref, h2_ref, bt_ref, lw1_ref, lb1_ref, lw2_ref, lb2_ref,
             o_ref):
        pre = t_ref[...] + a_ref[0, :_PR, :] + a_ref[1, :_PR, :] + b1_ref[...]
        m = _tile4(_fold4(jnp.mean(pre, axis=0, keepdims=True)))
        d = pre - m
        v = _tile4(_fold4(jnp.mean(d * d, axis=0, keepdims=True)))
        hn = d * (g_ref[...] * lax.rsqrt(v + 1e-5)) + be_ref[...]
        hn = _leaky(hn)
        h3 = _leaky(_dot(hn, w2_ref[...]) + b2_ref[...])
        seg = lax.broadcasted_iota(jnp.int32, (_G, _PR), 0)
        ps = []
        for hv in (h1_ref[...], h2_ref[...], h3):
            p = jnp.zeros((_G, _H), jnp.float32)
            for kk in range(4):
                onehot = (seg == bt_ref[kk, :]).astype(jnp.float32)
                p = p + _dot(onehot, hv[:, 32 * kk:32 * (kk + 1)])
            ps.append(p)
        p = jnp.concatenate(ps, axis=1)
        z = jnp.maximum(_dot(p, lw1_ref[...]) + lb1_ref[...], 0.0)
        z = _dot(z, lw2_ref[...]) + lb2_ref[...]
        o_ref[...] = _leaky(z)

    return pl.pallas_call(
        body,
        out_shape=jax.ShapeDtypeStruct((_G, _C), jnp.float32),
    )(t, agg, b1, gamma, beta, w2x4, b2, h1, h2, batch4, lw1, lb1, lw2, lb2)


def _blockdiag4(w):
    a, b = w.shape
    z = jnp.zeros((a, b), w.dtype)
    return jnp.block([[w if i == j else z for j in range(4)]
                      for i in range(4)])


def kernel(x, x_e, edge_index, batch, params):
    del x_e
    src = edge_index[0].astype(jnp.int32)
    dst = edge_index[1].astype(jnp.int32)
    npad = _EPAD - src.shape[0]
    src_w = jnp.pad(src, (0, npad)).reshape(_NW * _NCHUNK, _CHUNK)
    dst_w = jnp.pad(dst, (0, npad),
                    constant_values=_N).reshape(_NW * _NCHUNK, _CHUNK)
    z = jnp.zeros((_NA, _H), jnp.float32)
    batch4 = batch.astype(jnp.int32).reshape(4, _PR)

    p1, p2, p3 = params["conv1"], params["conv2"], params["conv3"]

    def r4(v):
        return jnp.tile(v.reshape(1, -1), (1, 4))

    t1 = _proj(x, p1["W1"])
    a1 = _segsum_sc(t1, src_w, dst_w, z)
    h1, t2 = _mlp_block(t1, a1, r4(p1["b1"]), r4(p1["gamma"]),
                        r4(p1["beta"]), _blockdiag4(p1["W2"]), r4(p1["b2"]),
                        _blockdiag4(p2["W1"]))
    a2 = _segsum_sc(t2, src_w, dst_w, z)
    h2, t3 = _mlp_block(t2, a2, r4(p2["b1"]), r4(p2["gamma"]),
                        r4(p2["beta"]), _blockdiag4(p2["W2"]), r4(p2["b2"]),
                        _blockdiag4(p3["W1"]))
    a3 = _segsum_sc(t3, src_w, dst_w, z)
    return _mlp3_head(t3, a3, r4(p3["b1"]), r4(p3["gamma"]), r4(p3["beta"]),
                      _blockdiag4(p3["W2"]), r4(p3["b2"]), h1, h2, batch4,
                      params["lin1"]["W"], params["lin1"]["b"].reshape(1, -1),
                      params["lin2"]["W"], params["lin2"]["b"].reshape(1, -1))
